# baseline (speedup 1.0000x reference)
import functools

import jax
import jax.numpy as jnp
from jax import lax
from jax.experimental import pallas as pl
from jax.experimental.pallas import tpu as pltpu
from jax.experimental.pallas import tpu_sc as plsc

NC = 2
NS = 16
NW = NC * NS
CH = 128


def _sc_mesh():
    return plsc.VectorSubcoreMesh(core_axis_name="c", subcore_axis_name="s")


def _make_hist(P, n_chunks_tile):
    rows_tile = P // NS

    @functools.partial(
        pl.kernel,
        out_type=jax.ShapeDtypeStruct((NC * P,), jnp.float32),
        mesh=_sc_mesh(),
        scratch_types=[
            pltpu.VMEM((n_chunks_tile, CH), jnp.int32),
            pltpu.VMEM((CH,), jnp.float32),
            pltpu.VMEM_SHARED((P,), jnp.float32),
        ],
    )
    def hist(dst_hbm, ones_hbm, zeros_hbm, out_hbm, idx_v, ones_v, hist_sh):
        c = lax.axis_index("c")
        s = lax.axis_index("s")
        w = s * NC + c
        pltpu.sync_copy(zeros_hbm, hist_sh.at[pl.ds(s * rows_tile, rows_tile)])
        pltpu.sync_copy(ones_hbm, ones_v)
        pltpu.sync_copy(dst_hbm.at[w], idx_v)
        plsc.subcore_barrier()

        def body(j, carry):
            pltpu.sync_copy(ones_v, hist_sh.at[idx_v.at[j]], add=True)
            return carry

        lax.fori_loop(0, n_chunks_tile, body, 0)
        plsc.subcore_barrier()
        pltpu.sync_copy(hist_sh.at[pl.ds(s * rows_tile, rows_tile)],
                        out_hbm.at[pl.ds(c * P + s * rows_tile, rows_tile)])

    return hist


def _make_scatter(P, n_chunks_tile, F, n_blk):
    rows_tile = P // NS
    blk = n_chunks_tile // n_blk

    @functools.partial(
        pl.kernel,
        out_type=jax.ShapeDtypeStruct((NC, P, F), jnp.float32),
        mesh=_sc_mesh(),
        scratch_types=[
            pltpu.VMEM((blk, CH), jnp.int32),
            pltpu.VMEM((blk, CH), jnp.int32),
            pltpu.VMEM((CH, F), jnp.float32),
            pltpu.VMEM((CH, F), jnp.float32),
            pltpu.VMEM_SHARED((P, F), jnp.float32),
            pltpu.SemaphoreType.DMA,
        ],
    )
    def scat(g_hbm, src_hbm, dst_hbm, out_hbm,
             sidx, didx, rows0, rows1, acc_sh, sem_g):
        c = lax.axis_index("c")
        s = lax.axis_index("s")
        w = s * NC + c
        bufs = (rows0, rows1)

        zvec = jnp.zeros((16,), jnp.float32)

        def zbody(r, carry):
            for cc in range(F // 16):
                rows0[r, pl.ds(cc * 16, 16)] = zvec
            return carry

        with jax.named_scope("acc_zero"):
            lax.fori_loop(0, CH, zbody, 0)
            for t in range(rows_tile // CH):
                pltpu.sync_copy(rows0,
                                acc_sh.at[pl.ds(s * rows_tile + t * CH, CH)])
            plsc.subcore_barrier()

        def gather(j, b):
            pltpu.make_async_copy(g_hbm.at[sidx.at[j]], bufs[b], sem_g).start()

        def wait_gather(b):
            pltpu.make_async_copy(g_hbm.at[sidx.at[0]], bufs[b], sem_g).wait()

        def scatter(j, b):
            pltpu.sync_copy(bufs[b], acc_sh.at[didx.at[j]], add=True)

        for ib in range(n_blk):
            with jax.named_scope("idx_load"):
                pltpu.sync_copy(src_hbm.at[w, pl.ds(ib * blk, blk)], sidx)
                pltpu.sync_copy(dst_hbm.at[w, pl.ds(ib * blk, blk)], didx)
            with jax.named_scope("edge_loop"):
                gather(0, 0)

                def body(k, carry):
                    j0 = 2 * k
                    wait_gather(0)
                    gather(j0 + 1, 1)
                    scatter(j0, 0)
                    wait_gather(1)

                    @pl.when(j0 + 2 < blk)
                    def _():
                        gather(j0 + 2, 0)

                    scatter(j0 + 1, 1)
                    return carry

                lax.fori_loop(0, blk // 2, body, 0)

        with jax.named_scope("writeout"):
            plsc.subcore_barrier()
            pltpu.sync_copy(acc_sh.at[pl.ds(s * rows_tile, rows_tile)],
                            out_hbm.at[c, pl.ds(s * rows_tile, rows_tile)])

    return scat


def _dinv_body(N, hist_ref, o_ref):
    R, C = o_ref.shape
    deg = hist_ref[0] + hist_ref[1] + 1.0
    flat = (lax.broadcasted_iota(jnp.int32, (R, C), 0) * C
            + lax.broadcasted_iota(jnp.int32, (R, C), 1))
    o_ref[...] = jnp.where(flat < N, lax.rsqrt(deg), 0.0)


def _mm1_body(dinv_ref, x_ref, w_ref, o_ref):
    o_ref[...] = dinv_ref[...] * jnp.dot(
        x_ref[...], w_ref[...], preferred_element_type=jnp.float32)


def _mm_body(dinv_ref, acc_ref, g_ref, b_ref, w_ref, o_ref):
    t = acc_ref[0] + acc_ref[1] + g_ref[...]
    h = jnp.maximum(dinv_ref[...] * t + b_ref[...], 0.0)
    o_ref[...] = dinv_ref[...] * jnp.dot(
        h, w_ref[...], preferred_element_type=jnp.float32)


def _ep_body(dinv_ref, acc_ref, g_ref, b_ref, o_ref):
    t = acc_ref[0] + acc_ref[1] + g_ref[...]
    o_ref[...] = jnp.maximum(dinv_ref[...] * t + b_ref[...], 0.0)


def kernel(x, edge_index, W1, b1, W2, b2, W3, b3, W4, b4):
    N, F = x.shape
    E = edge_index.shape[1]
    BR = 512
    N_BLK = 2
    P = -(-(N + 1) // BR) * BR
    n_chunks_tile = -(-E // (NW * CH * 2 * N_BLK)) * (2 * N_BLK)
    E_pad = n_chunks_tile * NW * CH
    rows_tile = P // NS
    grid = P // BR

    src = edge_index[0]
    dst = edge_index[1]
    fill_src = jnp.full((E_pad - E,), N, dtype=src.dtype)
    fill_dst = N + (jnp.arange(E_pad - E, dtype=dst.dtype) % (P - N))
    src_p = jnp.concatenate([src, fill_src]).reshape(NW, n_chunks_tile, CH)
    dst_p = jnp.concatenate([dst, fill_dst]).reshape(NW, n_chunks_tile, CH)
    x_p = jnp.zeros((P, F), jnp.float32).at[:N].set(x)
    zeros_1d = jnp.zeros((rows_tile,), jnp.float32)
    ones_ch = jnp.ones((CH,), jnp.float32)

    hist = _make_hist(P, n_chunks_tile)(dst_p, ones_ch, zeros_1d)
    scat = _make_scatter(P, n_chunks_tile, F, N_BLK)

    dinv2d = pl.pallas_call(
        functools.partial(_dinv_body, N),
        out_shape=jax.ShapeDtypeStruct((P // 128, 128), jnp.float32),
    )(hist.reshape(NC, P // 128, 128))
    dinv = dinv2d.reshape(P, 1)

    def mm1(dinv, xp, W):
        return pl.pallas_call(
            _mm1_body,
            grid=(grid,),
            in_specs=[
                pl.BlockSpec((BR, 1), lambda i: (i, 0)),
                pl.BlockSpec((BR, F), lambda i: (i, 0)),
                pl.BlockSpec((F, F), lambda i: (0, 0)),
            ],
            out_specs=pl.BlockSpec((BR, F), lambda i: (i, 0)),
            out_shape=jax.ShapeDtypeStruct((P, F), jnp.float32),
        )(dinv, xp, W)

    def mm(dinv, acc, g, b, W):
        return pl.pallas_call(
            _mm_body,
            grid=(grid,),
            in_specs=[
                pl.BlockSpec((BR, 1), lambda i: (i, 0)),
                pl.BlockSpec((NC, BR, F), lambda i: (0, i, 0)),
                pl.BlockSpec((BR, F), lambda i: (i, 0)),
                pl.BlockSpec((1, F), lambda i: (0, 0)),
                pl.BlockSpec((F, F), lambda i: (0, 0)),
            ],
            out_specs=pl.BlockSpec((BR, F), lambda i: (i, 0)),
            out_shape=jax.ShapeDtypeStruct((P, F), jnp.float32),
        )(dinv, acc, g, b.reshape(1, F), W)

    def ep(dinv, acc, g, b):
        return pl.pallas_call(
            _ep_body,
            grid=(grid,),
            in_specs=[
                pl.BlockSpec((BR, 1), lambda i: (i, 0)),
                pl.BlockSpec((NC, BR, F), lambda i: (0, i, 0)),
                pl.BlockSpec((BR, F), lambda i: (i, 0)),
                pl.BlockSpec((1, F), lambda i: (0, 0)),
            ],
            out_specs=pl.BlockSpec((BR, F), lambda i: (i, 0)),
            out_shape=jax.ShapeDtypeStruct((P, F), jnp.float32),
        )(dinv, acc, g, b.reshape(1, F))

    g = mm1(dinv, x_p, W1)
    acc = scat(g, src_p, dst_p)
    g = mm(dinv, acc, g, b1, W2)
    acc = scat(g, src_p, dst_p)
    g = mm(dinv, acc, g, b2, W3)
    acc = scat(g, src_p, dst_p)
    g = mm(dinv, acc, g, b3, W4)
    acc = scat(g, src_p, dst_p)
    out = ep(dinv, acc, g, b4)
    return out[:N]


# speedup vs baseline: 3.3454x; 3.3454x over previous
import functools

import jax
import jax.numpy as jnp
from jax import lax
from jax.experimental import pallas as pl
from jax.experimental.pallas import tpu as pltpu
from jax.experimental.pallas import tpu_sc as plsc

NC = 2
NS = 16
NW = NC * NS
CH = 128


def _sc_mesh():
    return plsc.VectorSubcoreMesh(core_axis_name="c", subcore_axis_name="s")


def _make_hist(P, n_chunks_tile, chunks_total):
    rows_tile = P // NS

    @functools.partial(
        pl.kernel,
        out_type=jax.ShapeDtypeStruct((NC * P,), jnp.float32),
        mesh=_sc_mesh(),
        scratch_types=[
            pltpu.VMEM((n_chunks_tile, CH), jnp.int32),
            pltpu.VMEM((CH,), jnp.float32),
            pltpu.VMEM_SHARED((P,), jnp.float32),
        ],
    )
    def hist(dst_hbm, ones_hbm, zeros_hbm, out_hbm, idx_v, ones_v, hist_sh):
        c = lax.axis_index("c")
        s = lax.axis_index("s")
        w = s * NC + c
        nch = (chunks_total - w + NW - 1) // NW
        pltpu.sync_copy(zeros_hbm, hist_sh.at[pl.ds(s * rows_tile, rows_tile)])
        pltpu.sync_copy(ones_hbm, ones_v)
        pltpu.sync_copy(dst_hbm.at[w], idx_v)
        plsc.subcore_barrier()

        def body(j, carry):
            pltpu.sync_copy(ones_v, hist_sh.at[idx_v.at[j]], add=True)
            return carry

        lax.fori_loop(0, nch, body, 0)
        plsc.subcore_barrier()
        pltpu.sync_copy(hist_sh.at[pl.ds(s * rows_tile, rows_tile)],
                        out_hbm.at[pl.ds(c * P + s * rows_tile, rows_tile)])

    return hist


def _make_scatter(P, n_chunks_tile, F, n_blk, chunks_total):
    rows_tile = P // NS
    blk = n_chunks_tile // n_blk

    @functools.partial(
        pl.kernel,
        out_type=jax.ShapeDtypeStruct((NC, P, F), jnp.float32),
        mesh=_sc_mesh(),
        scratch_types=[
            pltpu.VMEM((blk, CH), jnp.int32),
            pltpu.VMEM((blk, CH), jnp.int32),
            pltpu.VMEM((CH, F), jnp.float32),
            pltpu.VMEM((CH, F), jnp.float32),
            pltpu.VMEM_SHARED((P, F), jnp.float32),
            pltpu.SemaphoreType.DMA,
        ],
    )
    def scat(g_hbm, src_hbm, dst_hbm, out_hbm,
             sidx, didx, rows0, rows1, acc_sh, sem_g):
        c = lax.axis_index("c")
        s = lax.axis_index("s")
        w = s * NC + c
        nch = (chunks_total - w + NW - 1) // NW
        bufs = (rows0, rows1)

        zvec = jnp.zeros((16,), jnp.float32)

        def zbody(r, carry):
            for cc in range(F // 16):
                rows0[r, pl.ds(cc * 16, 16)] = zvec
            return carry

        with jax.named_scope("acc_zero"):
            lax.fori_loop(0, CH, zbody, 0)
            for t in range(rows_tile // CH):
                pltpu.sync_copy(rows0,
                                acc_sh.at[pl.ds(s * rows_tile + t * CH, CH)])
            plsc.subcore_barrier()

        def gather(j, b):
            pltpu.make_async_copy(g_hbm.at[sidx.at[j]], bufs[b], sem_g).start()

        def wait_gather(b):
            pltpu.make_async_copy(g_hbm.at[sidx.at[0]], bufs[b], sem_g).wait()

        def scatter(j, b):
            pltpu.sync_copy(bufs[b], acc_sh.at[didx.at[j]], add=True)

        for ib in range(n_blk):
            nb = jnp.clip(nch - ib * blk, 0, blk)
            with jax.named_scope("idx_load"):
                pltpu.sync_copy(src_hbm.at[w, pl.ds(ib * blk, blk)], sidx)
                pltpu.sync_copy(dst_hbm.at[w, pl.ds(ib * blk, blk)], didx)
            with jax.named_scope("edge_loop"):
                @pl.when(nb > 0)
                def _():
                    gather(0, 0)

                def body(k, carry):
                    j0 = 2 * k
                    wait_gather(0)

                    @pl.when(j0 + 1 < nb)
                    def _():
                        gather(j0 + 1, 1)

                    scatter(j0, 0)

                    @pl.when(j0 + 1 < nb)
                    def _():
                        wait_gather(1)

                        @pl.when(j0 + 2 < nb)
                        def _():
                            gather(j0 + 2, 0)

                        scatter(j0 + 1, 1)

                    return carry

                lax.fori_loop(0, (nb + 1) // 2, body, 0)

        with jax.named_scope("writeout"):
            plsc.subcore_barrier()
            pltpu.sync_copy(acc_sh.at[pl.ds(s * rows_tile, rows_tile)],
                            out_hbm.at[c, pl.ds(s * rows_tile, rows_tile)])

    return scat


def _dinv_body(N, hist_ref, o_ref):
    R, C = o_ref.shape
    deg = hist_ref[0] + hist_ref[1] + 1.0
    flat = (lax.broadcasted_iota(jnp.int32, (R, C), 0) * C
            + lax.broadcasted_iota(jnp.int32, (R, C), 1))
    o_ref[...] = jnp.where(flat < N, lax.rsqrt(deg), 0.0)


def _mm1_body(dinv_ref, x_ref, w_ref, o_ref):
    o_ref[...] = dinv_ref[...] * jnp.dot(
        x_ref[...], w_ref[...], preferred_element_type=jnp.float32)


def _mm_body(dinv_ref, acc_ref, g_ref, b_ref, w_ref, o_ref):
    t = acc_ref[0] + acc_ref[1] + g_ref[...]
    h = jnp.maximum(dinv_ref[...] * t + b_ref[...], 0.0)
    o_ref[...] = dinv_ref[...] * jnp.dot(
        h, w_ref[...], preferred_element_type=jnp.float32)


def _ep_body(dinv_ref, acc_ref, g_ref, b_ref, o_ref):
    t = acc_ref[0] + acc_ref[1] + g_ref[...]
    o_ref[...] = jnp.maximum(dinv_ref[...] * t + b_ref[...], 0.0)


def kernel(x, edge_index, W1, b1, W2, b2, W3, b3, W4, b4):
    N, F = x.shape
    E = edge_index.shape[1]
    BR = 512
    N_BLK = 2
    P = -(-(N + 1) // BR) * BR
    chunks_total = -(-E // CH)
    n_chunks_tile = -(-chunks_total // (NW * 2 * N_BLK)) * (2 * N_BLK)
    rows_tile = P // NS
    grid = P // BR

    src = edge_index[0]
    dst = edge_index[1]
    e_fill = chunks_total * CH - E
    c_fill = n_chunks_tile * NW - chunks_total
    fill_src = jnp.full((e_fill + c_fill * CH,), N, dtype=src.dtype)
    fill_dst = N + (jnp.arange(e_fill + c_fill * CH, dtype=dst.dtype)
                    % (P - N))
    src_p = (jnp.concatenate([src, fill_src])
             .reshape(n_chunks_tile, NW, CH).transpose(1, 0, 2))
    dst_p = (jnp.concatenate([dst, fill_dst])
             .reshape(n_chunks_tile, NW, CH).transpose(1, 0, 2))
    x_p = jnp.zeros((P, F), jnp.float32).at[:N].set(x)
    zeros_1d = jnp.zeros((rows_tile,), jnp.float32)
    ones_ch = jnp.ones((CH,), jnp.float32)

    hist = _make_hist(P, n_chunks_tile, chunks_total)(dst_p, ones_ch, zeros_1d)
    scat = _make_scatter(P, n_chunks_tile, F, N_BLK, chunks_total)

    dinv2d = pl.pallas_call(
        functools.partial(_dinv_body, N),
        out_shape=jax.ShapeDtypeStruct((P // 128, 128), jnp.float32),
    )(hist.reshape(NC, P // 128, 128))
    dinv = dinv2d.reshape(P, 1)

    def mm1(dinv, xp, W):
        return pl.pallas_call(
            _mm1_body,
            grid=(grid,),
            in_specs=[
                pl.BlockSpec((BR, 1), lambda i: (i, 0)),
                pl.BlockSpec((BR, F), lambda i: (i, 0)),
                pl.BlockSpec((F, F), lambda i: (0, 0)),
            ],
            out_specs=pl.BlockSpec((BR, F), lambda i: (i, 0)),
            out_shape=jax.ShapeDtypeStruct((P, F), jnp.float32),
        )(dinv, xp, W)

    def mm(dinv, acc, g, b, W):
        return pl.pallas_call(
            _mm_body,
            grid=(grid,),
            in_specs=[
                pl.BlockSpec((BR, 1), lambda i: (i, 0)),
                pl.BlockSpec((NC, BR, F), lambda i: (0, i, 0)),
                pl.BlockSpec((BR, F), lambda i: (i, 0)),
                pl.BlockSpec((1, F), lambda i: (0, 0)),
                pl.BlockSpec((F, F), lambda i: (0, 0)),
            ],
            out_specs=pl.BlockSpec((BR, F), lambda i: (i, 0)),
            out_shape=jax.ShapeDtypeStruct((P, F), jnp.float32),
        )(dinv, acc, g, b.reshape(1, F), W)

    def ep(dinv, acc, g, b):
        return pl.pallas_call(
            _ep_body,
            grid=(grid,),
            in_specs=[
                pl.BlockSpec((BR, 1), lambda i: (i, 0)),
                pl.BlockSpec((NC, BR, F), lambda i: (0, i, 0)),
                pl.BlockSpec((BR, F), lambda i: (i, 0)),
                pl.BlockSpec((1, F), lambda i: (0, 0)),
            ],
            out_specs=pl.BlockSpec((BR, F), lambda i: (i, 0)),
            out_shape=jax.ShapeDtypeStruct((P, F), jnp.float32),
        )(dinv, acc, g, b.reshape(1, F))

    g = mm1(dinv, x_p, W1)
    acc = scat(g, src_p, dst_p)
    g = mm(dinv, acc, g, b1, W2)
    acc = scat(g, src_p, dst_p)
    g = mm(dinv, acc, g, b2, W3)
    acc = scat(g, src_p, dst_p)
    g = mm(dinv, acc, g, b3, W4)
    acc = scat(g, src_p, dst_p)
    out = ep(dinv, acc, g, b4)
    return out[:N]
